# baseline (speedup 1.0000x reference)
import functools

import jax
import jax.numpy as jnp
from jax import lax
from jax.experimental import pallas as pl
from jax.experimental.pallas import tpu as pltpu
from jax.experimental.pallas import tpu_sc as plsc

D = 64
EPS = 1e-5

GV = 128
GPC = 8
CHUNK = GV * GPC


def _gather_sc(table, idx2d):
    n_idx_rows = idx2d.shape[0]
    n = n_idx_rows * GV
    info = plsc.get_sparse_core_info()
    nc, ns = info.num_cores, info.num_subcores
    nw = nc * ns
    n_per_w = n // nw
    n_chunks = n_per_w // CHUNK
    assert n_per_w % CHUNK == 0

    mesh = plsc.VectorSubcoreMesh(core_axis_name="c", subcore_axis_name="s")

    @functools.partial(
        pl.kernel,
        mesh=mesh,
        out_type=jax.ShapeDtypeStruct((n, D), jnp.float32),
        compiler_params=pltpu.CompilerParams(use_tc_tiling_on_sc=False),
        scratch_types=[
            pltpu.VMEM((GPC, GV), jnp.int32),
            pltpu.VMEM((CHUNK, D), jnp.float32),
            pltpu.SemaphoreType.DMA,
        ],
    )
    def k(table_hbm, idx_hbm, out_hbm, idx_v, rows_v, gsem):
        wid = lax.axis_index("s") * nc + lax.axis_index("c")
        row_base = wid * n_per_w

        def chunk_body(i, carry):
            off = pl.multiple_of(row_base + i * CHUNK, CHUNK)
            pltpu.sync_copy(
                idx_hbm.at[pl.ds(pl.multiple_of(off // GV, GPC), GPC)], idx_v
            )
            for j in range(GPC):
                pltpu.async_copy(
                    table_hbm.at[idx_v.at[j]],
                    rows_v.at[pl.ds(j * GV, GV)],
                    gsem,
                )
            for j in range(GPC):
                pltpu.make_async_copy(
                    table_hbm.at[idx_v.at[j]],
                    rows_v.at[pl.ds(j * GV, GV)],
                    gsem,
                ).wait()
            pltpu.sync_copy(rows_v, out_hbm.at[pl.ds(off, CHUNK)])
            return carry

        lax.fori_loop(0, n_chunks, chunk_body, 0)

    return k(table, idx2d)


def _ln_tc(rows, gamma, beta):
    n = rows.shape[0]
    blk = 4096
    grid = n // blk

    def body(r_ref, g_ref, b_ref, o_ref):
        v = r_ref[...]
        mean = jnp.mean(v, axis=-1, keepdims=True)
        c = v - mean
        var = jnp.mean(c * c, axis=-1, keepdims=True)
        o_ref[...] = c * lax.rsqrt(var + EPS) * g_ref[...] + b_ref[...]

    return pl.pallas_call(
        body,
        grid=(grid,),
        in_specs=[
            pl.BlockSpec((blk, D), lambda i: (i, 0)),
            pl.BlockSpec((1, D), lambda i: (0, 0)),
            pl.BlockSpec((1, D), lambda i: (0, 0)),
        ],
        out_specs=pl.BlockSpec((blk, D), lambda i: (i, 0)),
        out_shape=jax.ShapeDtypeStruct((n, D), jnp.float32),
    )(rows, gamma.reshape(1, D), beta.reshape(1, D))


def kernel(x, table, gamma, beta):
    b, s = x.shape
    idx2d = x.reshape(-1, GV).astype(jnp.int32)
    rows = _gather_sc(table, idx2d)
    out = _ln_tc(rows, gamma, beta)
    return out.reshape(b, s, D)


# speedup vs baseline: 1.3866x; 1.3866x over previous
import functools

import jax
import jax.numpy as jnp
from jax import lax
from jax.experimental import pallas as pl
from jax.experimental.pallas import tpu as pltpu
from jax.experimental.pallas import tpu_sc as plsc

D = 64
EPS = 1e-5

GV = 128
GPC = 8
CHUNK = GV * GPC


def _gather_sc(table_rows, idx):
    n = idx.shape[0]
    info = plsc.get_sparse_core_info()
    nc, ns = info.num_cores, info.num_subcores
    nw = nc * ns
    n_per_w = n // nw
    n_chunks = n_per_w // CHUNK
    assert n_per_w % CHUNK == 0

    mesh = plsc.VectorSubcoreMesh(core_axis_name="c", subcore_axis_name="s")

    @functools.partial(
        pl.kernel,
        mesh=mesh,
        out_type=jax.ShapeDtypeStruct((n, D), jnp.float32),
        compiler_params=pltpu.CompilerParams(use_tc_tiling_on_sc=False),
        scratch_types=[
            pltpu.VMEM((CHUNK,), jnp.int32),
            pltpu.VMEM((CHUNK, D), jnp.float32),
            pltpu.SemaphoreType.DMA,
        ],
    )
    def k(table_hbm, idx_hbm, out_hbm, idx_v, rows_v, gsem):
        wid = lax.axis_index("s") * nc + lax.axis_index("c")
        row_base = wid * n_per_w

        def chunk_body(i, carry):
            off = pl.multiple_of(row_base + i * CHUNK, CHUNK)
            pltpu.sync_copy(idx_hbm.at[pl.ds(off, CHUNK)], idx_v)
            for j in range(GPC):
                pltpu.async_copy(
                    table_hbm.at[idx_v.at[pl.ds(j * GV, GV)]],
                    rows_v.at[pl.ds(j * GV, GV)],
                    gsem,
                )
            for j in range(GPC):
                pltpu.make_async_copy(
                    table_hbm.at[idx_v.at[pl.ds(j * GV, GV)]],
                    rows_v.at[pl.ds(j * GV, GV)],
                    gsem,
                ).wait()
            pltpu.sync_copy(rows_v, out_hbm.at[pl.ds(off, CHUNK)])
            return carry

        lax.fori_loop(0, n_chunks, chunk_body, 0)

    return k(table_rows, idx)


def _ln_tc(rows2, gamma_t, beta_t, s, b):
    h = b // 2

    def body(r_ref, g_ref, b_ref, o_ref):
        w = jnp.transpose(r_ref[...])
        u = jnp.concatenate([w[:D, :], w[D:, :]], axis=1)
        mean = jnp.mean(u, axis=0, keepdims=True)
        c = u - mean
        var = jnp.mean(c * c, axis=0, keepdims=True)
        o_ref[...] = (c * lax.rsqrt(var + EPS) * g_ref[...] + b_ref[...])[None]

    return pl.pallas_call(
        body,
        grid=(s,),
        in_specs=[
            pl.BlockSpec((h, 2 * D), lambda i: (i, 0)),
            pl.BlockSpec((D, 1), lambda i: (0, 0)),
            pl.BlockSpec((D, 1), lambda i: (0, 0)),
        ],
        out_specs=pl.BlockSpec((1, D, b), lambda i: (i, 0, 0)),
        out_shape=jax.ShapeDtypeStruct((s, D, b), jnp.float32),
    )(rows2, gamma_t, beta_t)


def kernel(x, table, gamma, beta):
    b, s = x.shape
    v = table.shape[0]
    tt3 = jnp.transpose(table).reshape(D, v // 2, 2)
    t2 = jnp.transpose(tt3, (1, 2, 0))
    table_rows = t2.reshape(v, D)
    xt3 = jnp.transpose(x).reshape(s, 2, b // 2)
    idx = jnp.transpose(xt3, (0, 2, 1)).reshape(-1).astype(jnp.int32)
    rows = _gather_sc(table_rows, idx)
    rows2 = rows.reshape((b * s) // 2, 2 * D)
    y = _ln_tc(rows2, gamma.reshape(D, 1), beta.reshape(D, 1), s, b)
    return jnp.transpose(y, (2, 0, 1))
